# baseline (speedup 1.0000x reference)
import functools

import jax
import jax.numpy as jnp
from jax import lax
from jax.experimental import pallas as pl
from jax.experimental.pallas import tpu as pltpu
from jax.experimental.pallas import tpu_sc as plsc

_N_CLASSES = 1000
_CTX_LEN = 77
_N_CTX = 4
_D = 768
_SUF = _CTX_LEN - 1 - _N_CTX
_LANES = 16
_NG = _D // _LANES

_NC = 2
_NS = 16

_BIG_W = 29
_BIG_N = 32
_SMALL_N = 24
_PREF_BLK = 8


def _copy_row(dst, dst_r, src, src_r):
    for k in range(_NG):
        dst[dst_r, pl.ds(_LANES * k, _LANES)] = src[src_r, pl.ds(_LANES * k, _LANES)]


def _body(prefix_hbm, ctx_hbm, suffix_hbm, out_hbm, stage, frame, pref_buf,
          ctx_buf, sem_in, sem_pre, sem_out):
    wid = lax.axis_index("s") * _NC + lax.axis_index("c")
    big = wid < _BIG_W
    n = jnp.where(big, _BIG_N, _SMALL_N)
    base = jnp.where(
        big, wid * _BIG_N, _BIG_W * _BIG_N + (wid - _BIG_W) * _SMALL_N
    )

    pltpu.async_copy(ctx_hbm, ctx_buf, sem_in).wait()
    for r in range(_N_CTX):
        _copy_row(frame, 1 + r, ctx_buf, r)

    pltpu.async_copy(suffix_hbm.at[base], stage, sem_in)

    def cls(i, carry):
        c = base + i

        @pl.when(lax.bitwise_and(i, _PREF_BLK - 1) == 0)
        def _():
            blk = pl.multiple_of(
                lax.bitwise_and(c, jnp.int32(~(_PREF_BLK - 1))), _PREF_BLK
            )
            pltpu.async_copy(
                prefix_hbm.at[pl.ds(blk, _PREF_BLK)], pref_buf, sem_pre
            ).wait()

        pltpu.make_async_copy(suffix_hbm.at[c], stage, sem_in).wait()

        @pl.when(i > 0)
        def _():
            pltpu.make_async_copy(frame, out_hbm.at[c], sem_out).wait()

        _copy_row(frame, 0, pref_buf, lax.bitwise_and(i, _PREF_BLK - 1))

        def shift(r, carry2):
            _copy_row(frame, 1 + _N_CTX + r, stage, r)
            return carry2

        lax.fori_loop(0, _SUF, shift, 0)

        pltpu.async_copy(frame, out_hbm.at[c], sem_out)

        @pl.when(i + 1 < n)
        def _():
            pltpu.async_copy(suffix_hbm.at[c + 1], stage, sem_in)

        return carry

    lax.fori_loop(0, n, cls, 0)

    pltpu.make_async_copy(frame, out_hbm.at[base + n - 1], sem_out).wait()


@jax.jit
def _prompt_concat(token_prefix, ctx_embedding, token_suffix):
    run = functools.partial(
        pl.kernel,
        out_type=jax.ShapeDtypeStruct((_N_CLASSES, _CTX_LEN, _D), jnp.float32),
        mesh=plsc.VectorSubcoreMesh(core_axis_name="c", subcore_axis_name="s"),
        scratch_types=[
            pltpu.VMEM((_SUF, _D), jnp.float32),
            pltpu.VMEM((_CTX_LEN, _D), jnp.float32),
            pltpu.VMEM((_PREF_BLK, _D), jnp.float32),
            pltpu.VMEM((_N_CTX, _D), jnp.float32),
            pltpu.SemaphoreType.DMA,
            pltpu.SemaphoreType.DMA,
            pltpu.SemaphoreType.DMA,
        ],
    )(_body)
    return run(token_prefix.reshape(_N_CLASSES, _D), ctx_embedding, token_suffix)


def kernel(token_prefix, ctx_embedding, token_suffix, eos_position):
    prompts = _prompt_concat(token_prefix, ctx_embedding, token_suffix)
    return (prompts, eos_position)


# speedup vs baseline: 2.2620x; 2.2620x over previous
import functools

import jax
import jax.numpy as jnp
from jax.experimental import pallas as pl
from jax.experimental.pallas import tpu as pltpu

_N_CLASSES = 1000
_CTX_LEN = 77
_N_CTX = 4
_D = 768
_SUF = _CTX_LEN - 1 - _N_CTX

_CB = 40


def _body(prefix_ref, ctx_ref, suffix_ref, out_ref):
    out_ref[:, 0:1, :] = prefix_ref[...]
    out_ref[:, 1 : 1 + _N_CTX, :] = jnp.broadcast_to(
        ctx_ref[...][None], (_CB, _N_CTX, _D)
    )
    out_ref[:, 1 + _N_CTX :, :] = suffix_ref[...]


@jax.jit
def _prompt_concat(token_prefix, ctx_embedding, token_suffix):
    grid = (_N_CLASSES // _CB,)
    return pl.pallas_call(
        _body,
        grid=grid,
        in_specs=[
            pl.BlockSpec((_CB, 1, _D), lambda i: (i, 0, 0)),
            pl.BlockSpec((_N_CTX, _D), lambda i: (0, 0)),
            pl.BlockSpec((_CB, _SUF, _D), lambda i: (i, 0, 0)),
        ],
        out_specs=pl.BlockSpec((_CB, _CTX_LEN, _D), lambda i: (i, 0, 0)),
        out_shape=jax.ShapeDtypeStruct((_N_CLASSES, _CTX_LEN, _D), jnp.float32),
        compiler_params=pltpu.CompilerParams(
            dimension_semantics=("arbitrary",),
        ),
    )(token_prefix, ctx_embedding, token_suffix)


def kernel(token_prefix, ctx_embedding, token_suffix, eos_position):
    prompts = _prompt_concat(token_prefix, ctx_embedding, token_suffix)
    return (prompts, eos_position)
